# baseline (speedup 1.0000x reference)
import functools

import jax
import jax.numpy as jnp
from jax import lax
from jax.experimental import pallas as pl
from jax.experimental.pallas import tpu as pltpu
from jax.experimental.pallas import tpu_sc as plsc

NC = 2
NS = 16
NW = NC * NS
B = 16384
D = 16
BPW = B // NW
NG = BPW // 16
V = 1_000_000
VR = V * D // 128
VF = 7812 * 128
TB = 12
TCOLS = VF // TB
TROWS = TCOLS * D // 128


@functools.cache
def _mesh():
    return plsc.VectorSubcoreMesh(
        core_axis_name="c", subcore_axis_name="s", num_cores=NC, num_subcores=NS
    )


def _repack_body(x_hbm, tail_ref, o_hbm, x_v, o_v, tail_v, sem_in, sem_out):
    b = pl.program_id(0)
    cp_in = pltpu.make_async_copy(
        x_hbm.at[:, pl.ds(b * TCOLS, TCOLS)], x_v, sem_in)
    cp_in.start()
    cp_in.wait()
    z = x_v[...].T.reshape(TROWS, 8, D)
    o_v[...] = jnp.concatenate([z[:, e, :] for e in range(8)], axis=1)
    cp_out = pltpu.make_async_copy(
        o_v, o_hbm.at[pl.ds(b * TROWS, TROWS), :], sem_out)
    cp_out.start()
    cp_out.wait()

    @pl.when(b == 0)
    def _():
        tail_v[...] = tail_ref[...].T.reshape(8, 128)
        cp_t = pltpu.make_async_copy(
            tail_v, o_hbm.at[pl.ds(VF * D // 128, 8), :], sem_out)
        cp_t.start()
        cp_t.wait()


def _repack_tc(tableT, tail):
    return pl.pallas_call(
        _repack_body,
        grid=(TB,),
        in_specs=[
            pl.BlockSpec(memory_space=pl.ANY),
            pl.BlockSpec((D, V - VF), lambda b: (0, 0)),
        ],
        out_specs=pl.BlockSpec(memory_space=pl.ANY),
        out_shape=jax.ShapeDtypeStruct((VR, 128), jnp.float32),
        scratch_shapes=[
            pltpu.VMEM((D, TCOLS), jnp.float32),
            pltpu.VMEM((TROWS, 128), jnp.float32),
            pltpu.VMEM((8, 128), jnp.float32),
            pltpu.SemaphoreType.DMA,
            pltpu.SemaphoreType.DMA,
        ],
    )(tableT, tail)


def _sc_body(idx_hbm, node_p, rela_p, link_p, err_hbm,
             idx_v, s0, s1, s2, s3, s4, s5, tbuf, out_v, sem,
             j0, j1, j2, j3, j4, j5):
    wid = lax.axis_index("s") * NC + lax.axis_index("c")
    pltpu.sync_copy(idx_hbm.at[wid], idx_v)

    tabs = (rela_p, link_p, node_p, node_p, node_p, node_p)
    cols = (0, 0, 1, 2, 3, 4)
    slabs = (s0, s1, s2, s3, s4, s5)
    iota = lax.iota(jnp.int32, 16)

    def group(g, carry):
        base = g * 16
        vecs = [
            plsc.load_gather(
                idx_v,
                [(c * BPW + base + iota) >> 7, (c * BPW + base + iota) & 127],
            )
            for c in range(5)
        ]
        jrefs = (j0, j1, j2, j3, j4, j5)
        for t in range(6):
            jrefs[t][...] = vecs[cols[t]] >> 3
        copies = [
            pltpu.async_copy(tabs[t].at[jrefs[t]], slabs[t], sem)
            for t in range(6)
        ]
        for c in copies:
            c.wait()
        for l in range(16):
            row = jnp.full((16,), l, jnp.int32)
            sub = [(vecs[c][l] & 7) * 16 + iota for c in range(5)]
            rk = plsc.load_gather(s0, [row, sub[0]])
            lk = plsc.load_gather(s1, [row, sub[0]])
            pi = plsc.load_gather(s2, [row, sub[1]])
            pj = plsc.load_gather(s3, [row, sub[2]])
            ni = plsc.load_gather(s4, [row, sub[3]])
            nj = plsc.load_gather(s5, [row, sub[4]])
            t_ = (nj - ni - pj + pi) * rk + (ni * nj - pi * pj) * lk
            plsc.store_scatter(tbuf, [iota * 16 + l], t_)
        acc = jnp.zeros((16,), jnp.float32)
        for d in range(16):
            acc = acc + plsc.load_gather(tbuf, [d * 16 + iota])
        flat = base + iota
        plsc.store_scatter(out_v, [flat >> 7, flat & 127], acc)
        return carry

    lax.fori_loop(0, NG, group, 0)
    pltpu.sync_copy(out_v, err_hbm.at[wid])


@functools.cache
def _sc_err(interpret=False):
    return pl.kernel(
        _sc_body,
        out_type=jax.ShapeDtypeStruct((NW, 4, 128), jnp.float32),
        mesh=_mesh(),
        scratch_types=[
            pltpu.VMEM((5 * BPW // 128, 128), jnp.int32),
            pltpu.VMEM((16, 128), jnp.float32),
            pltpu.VMEM((16, 128), jnp.float32),
            pltpu.VMEM((16, 128), jnp.float32),
            pltpu.VMEM((16, 128), jnp.float32),
            pltpu.VMEM((16, 128), jnp.float32),
            pltpu.VMEM((16, 128), jnp.float32),
            pltpu.VMEM((256,), jnp.float32),
            pltpu.VMEM((4, 128), jnp.float32),
            pltpu.SemaphoreType.DMA,
            pltpu.VMEM((16,), jnp.int32),
            pltpu.VMEM((16,), jnp.int32),
            pltpu.VMEM((16,), jnp.int32),
            pltpu.VMEM((16,), jnp.int32),
            pltpu.VMEM((16,), jnp.int32),
            pltpu.VMEM((16,), jnp.int32),
        ],
        compiler_params=pltpu.CompilerParams(needs_layout_passes=False),
        interpret=interpret,
    )


def _softplus_body(x_ref, o_ref):
    v = x_ref[...]
    o_ref[...] = jnp.maximum(v, 0.0) + jnp.log1p(jnp.exp(-jnp.abs(v)))


def _softplus_tc(err):
    x = err.reshape(128, 128)
    y = pl.pallas_call(
        _softplus_body,
        out_shape=jax.ShapeDtypeStruct((128, 128), jnp.float32),
    )(x)
    return y.reshape(B)


def kernel(batchVector, nodeEmbedding, relaEmbedding, linkEmbedding):
    idx = (batchVector.astype(jnp.int32)
           .reshape(NW, BPW, 5)
           .transpose(0, 2, 1)
           .reshape(NW, 5 * BPW // 128, 128))
    node_p = nodeEmbedding.reshape(VR, 128)
    rela_p = relaEmbedding.reshape(VR, 128)
    link_p = linkEmbedding.reshape(VR, 128)
    err = _sc_err()(idx, node_p, rela_p, link_p)
    return _softplus_tc(err.reshape(B))


# speedup vs baseline: 1.0322x; 1.0322x over previous
import functools

import jax
import jax.numpy as jnp
from jax import lax
from jax.experimental import pallas as pl
from jax.experimental.pallas import tpu as pltpu
from jax.experimental.pallas import tpu_sc as plsc

NC = 2
NS = 16
NW = NC * NS
B = 16384
D = 16
BPW = B // NW
CH = 128
NCHUNK = BPW // CH

@functools.cache
def _mesh():
    return plsc.VectorSubcoreMesh(
        core_axis_name="c", subcore_axis_name="s", num_cores=NC, num_subcores=NS
    )


def _compute_groups(rela_v, link_v, pi_v, pj_v, ni_v, nj_v, tbuf, out_v):
    iota = lax.iota(jnp.int32, 16)

    def group(g, carry):
        for r in range(16):
            b = g * 16 + r
            pi = pi_v[b]
            pj = pj_v[b]
            ni = ni_v[b]
            nj = nj_v[b]
            rk = rela_v[b]
            lk = link_v[b]
            t = (nj - ni - pj + pi) * rk + (ni * nj - pi * pj) * lk
            plsc.store_scatter(tbuf, [iota, jnp.full((16,), r, jnp.int32)], t)
        acc = tbuf[0]
        for r in range(1, 16):
            acc = acc + tbuf[r]
        out_v[pl.ds(g * 16, 16)] = acc
        return carry

    lax.fori_loop(0, BPW // 16, group, 0)


def _sc_body(idx_hbm, node_hbm, rela_hbm, link_hbm, err_hbm,
             idx_v, rela_v, link_v, pi_v, pj_v, ni_v, nj_v, tbuf, out_v, sem):
    wid = lax.axis_index("s") * NC + lax.axis_index("c")
    pltpu.sync_copy(idx_hbm.at[wid], idx_v)

    copies = []
    for j in range(NCHUNK):
        sl = pl.ds(j * CH, CH)
        for col, table, dest in (
            (0, rela_hbm, rela_v), (0, link_hbm, link_v),
            (1, node_hbm, pi_v), (2, node_hbm, pj_v),
            (3, node_hbm, ni_v), (4, node_hbm, nj_v),
        ):
            copies.append(
                pltpu.async_copy(table.at[idx_v.at[col, j]], dest.at[sl], sem))
    for c in copies:
        c.wait()

    _compute_groups(rela_v, link_v, pi_v, pj_v, ni_v, nj_v, tbuf, out_v)
    pltpu.sync_copy(out_v, err_hbm.at[pl.ds(wid * BPW, BPW)])


@functools.cache
def _sc_err():
    return pl.kernel(
        _sc_body,
        out_type=jax.ShapeDtypeStruct((B,), jnp.float32),
        mesh=_mesh(),
        scratch_types=[
            pltpu.VMEM((5, NCHUNK, CH), jnp.int32),
            pltpu.VMEM((BPW, D), jnp.float32),
            pltpu.VMEM((BPW, D), jnp.float32),
            pltpu.VMEM((BPW, D), jnp.float32),
            pltpu.VMEM((BPW, D), jnp.float32),
            pltpu.VMEM((BPW, D), jnp.float32),
            pltpu.VMEM((BPW, D), jnp.float32),
            pltpu.VMEM((16, 16), jnp.float32),
            pltpu.VMEM((BPW,), jnp.float32),
            pltpu.SemaphoreType.DMA,
        ],
        compiler_params=pltpu.CompilerParams(
            needs_layout_passes=False, use_tc_tiling_on_sc=False),
    )


def _softplus_body(x_ref, o_ref):
    v = x_ref[...]
    o_ref[...] = jnp.maximum(v, 0.0) + jnp.log1p(jnp.exp(-jnp.abs(v)))


def _softplus_tc(err):
    x = err.reshape(128, 128)
    y = pl.pallas_call(
        _softplus_body,
        out_shape=jax.ShapeDtypeStruct((128, 128), jnp.float32),
    )(x)
    return y.reshape(B)


def kernel(batchVector, nodeEmbedding, relaEmbedding, linkEmbedding):
    idx = (batchVector.astype(jnp.int32)
           .reshape(NW, BPW, 5)
           .transpose(0, 2, 1)
           .reshape(NW, 5, NCHUNK, CH))
    err = _sc_err()(idx, nodeEmbedding, relaEmbedding, linkEmbedding)
    return _softplus_tc(err)


# speedup vs baseline: 3.4432x; 3.3357x over previous
import functools

import jax
import jax.numpy as jnp
from jax import lax
from jax.experimental import pallas as pl
from jax.experimental.pallas import tpu as pltpu
from jax.experimental.pallas import tpu_sc as plsc

NC = 2
NS = 16
NW = NC * NS
B = 16384
D = 16
BPW = B // NW
NG = BPW // 16


@functools.cache
def _mesh():
    return plsc.VectorSubcoreMesh(
        core_axis_name="c", subcore_axis_name="s", num_cores=NC, num_subcores=NS
    )


def _sc_body(idx_hbm, nodeT, relaT, linkT, err_hbm, *scratch):
    idx_v = scratch[0]
    tbuf = scratch[1]
    out_v = scratch[2]
    sem = scratch[3]
    slabs = scratch[4:]
    wid = lax.axis_index("s") * NC + lax.axis_index("c")
    pltpu.sync_copy(idx_hbm.at[wid], idx_v)

    tabs = (relaT, linkT, nodeT, nodeT, nodeT, nodeT)
    cols = (0, 0, 1, 2, 3, 4)
    iota = lax.iota(jnp.int32, 16)

    def group(g, carry):
        base = g * 16
        vecs = [
            plsc.load_gather(
                idx_v,
                [(c * BPW + base + iota) >> 7, (c * BPW + base + iota) & 127],
            )
            for c in range(5)
        ]
        for half in range(2):
            copies = []
            for l8 in range(8):
                l = half * 8 + l8
                for t in range(6):
                    i = vecs[cols[t]][l]
                    off = pl.multiple_of((i >> 7) * 128, 128)
                    copies.append(pltpu.async_copy(
                        tabs[t].at[:, pl.ds(off, 128)], slabs[l8 * 6 + t], sem))
            for c in copies:
                c.wait()
            for l8 in range(8):
                l = half * 8 + l8
                sub = [jnp.full((16,), vecs[c][l] & 127, jnp.int32)
                       for c in range(5)]
                rk = plsc.load_gather(slabs[l8 * 6 + 0], [iota, sub[0]])
                lk = plsc.load_gather(slabs[l8 * 6 + 1], [iota, sub[0]])
                pi = plsc.load_gather(slabs[l8 * 6 + 2], [iota, sub[1]])
                pj = plsc.load_gather(slabs[l8 * 6 + 3], [iota, sub[2]])
                ni = plsc.load_gather(slabs[l8 * 6 + 4], [iota, sub[3]])
                nj = plsc.load_gather(slabs[l8 * 6 + 5], [iota, sub[4]])
                t_ = (nj - ni - pj + pi) * rk + (ni * nj - pi * pj) * lk
                plsc.store_scatter(tbuf, [iota * 16 + l], t_)
        acc = jnp.zeros((16,), jnp.float32)
        for d in range(16):
            acc = acc + plsc.load_gather(tbuf, [d * 16 + iota])
        flat = base + iota
        plsc.store_scatter(out_v, [flat >> 7, flat & 127], acc)
        return carry

    lax.fori_loop(0, NG, group, 0)
    pltpu.sync_copy(out_v, err_hbm.at[wid])


@functools.cache
def _sc_err():
    return pl.kernel(
        _sc_body,
        out_type=jax.ShapeDtypeStruct((NW, 4, 128), jnp.float32),
        mesh=_mesh(),
        scratch_types=[
            pltpu.VMEM((5 * BPW // 128, 128), jnp.int32),
            pltpu.VMEM((256,), jnp.float32),
            pltpu.VMEM((4, 128), jnp.float32),
            pltpu.SemaphoreType.DMA,
        ] + [pltpu.VMEM((D, 128), jnp.float32)] * 48,
        compiler_params=pltpu.CompilerParams(needs_layout_passes=False),
    )


def _softplus_body(x_ref, o_ref):
    v = x_ref[...]
    o_ref[...] = jnp.maximum(v, 0.0) + jnp.log1p(jnp.exp(-jnp.abs(v)))


def _softplus_tc(err):
    x = err.reshape(128, 128)
    y = pl.pallas_call(
        _softplus_body,
        out_shape=jax.ShapeDtypeStruct((128, 128), jnp.float32),
    )(x)
    return y.reshape(B)


def kernel(batchVector, nodeEmbedding, relaEmbedding, linkEmbedding):
    idx = (batchVector.astype(jnp.int32)
           .reshape(NW, BPW, 5)
           .transpose(0, 2, 1)
           .reshape(NW, 5 * BPW // 128, 128))
    err = _sc_err()(idx, nodeEmbedding.T, relaEmbedding.T, linkEmbedding.T)
    return _softplus_tc(err.reshape(B))
